# baseline (speedup 1.0000x reference)
import jax
import jax.numpy as jnp
from jax.experimental import pallas as pl
from jax.experimental.pallas import tpu as pltpu

_M = 35
_D = 768
_H = _D // 2
_MOM = 0.8
_TBLK = 8192
_NCHUNK = 4


def _row_to_col(v, m):
    i0 = jax.lax.broadcasted_iota(jnp.int32, (m, m), 0)
    i1 = jax.lax.broadcasted_iota(jnp.int32, (m, m), 1)
    sel = jnp.where(i0 == i1, jnp.broadcast_to(v, (m, m)), 0.0)
    return jnp.sum(sel, axis=1, keepdims=True)


def _fused_kernel(x_ref, glove_ref, lnw_ref, lnb_ref, w1_ref, b1_ref,
                  w2_ref, b2_ref, out_ref, c_ref, cnt_ref, acc_ref, ng_ref):
    i = pl.program_id(0)
    nblk = pl.num_programs(0)

    @pl.when(i == 0)
    def _init():
        g = glove_ref[...]
        gn = jnp.sqrt(jnp.sum(g * g, axis=1, keepdims=True))
        ng_ref[...] = g / jnp.maximum(gn, 1e-12)
        c_ref[...] = jnp.full_like(c_ref, -2.0)
        cnt_ref[...] = jnp.zeros_like(cnt_ref)
        acc_ref[...] = jnp.zeros_like(acc_ref)

    ng = ng_ref[...]
    half = _TBLK // _NCHUNK
    accs, cnts, cs = [], [], []
    for h in range(_NCHUNK):
        x = x_ref[h * half:(h + 1) * half, :]
        ss = x[:, 0:128] * x[:, 0:128]
        for k in range(1, _D // 128):
            xs = x[:, k * 128:(k + 1) * 128]
            ss = ss + xs * xs
        sumsq = jnp.sum(ss, axis=1, keepdims=True)
        inv = jax.lax.rsqrt(jnp.maximum(sumsq, 1e-24))
        u = jax.lax.dot_general(x, ng, (((1,), (1,)), ((), ())),
                                preferred_element_type=jnp.float32)
        s = u * inv
        rowmax = jnp.max(s, axis=1, keepdims=True)
        onehot = jnp.where(s == rowmax, 1.0, 0.0)
        w = onehot * jnp.exp(rowmax)
        accs.append(jax.lax.dot_general(w, x, (((0,), (0,)), ((), ())),
                                        preferred_element_type=jnp.float32))
        cnts.append(jnp.sum(onehot, axis=0, keepdims=True))
        cs.append(jnp.max(s, axis=0, keepdims=True))
    acc_tot, cnt_tot, c_tot = accs[0], cnts[0], cs[0]
    for h in range(1, _NCHUNK):
        acc_tot = acc_tot + accs[h]
        cnt_tot = cnt_tot + cnts[h]
        c_tot = jnp.maximum(c_tot, cs[h])
    acc_ref[...] += acc_tot
    cnt_ref[...] += cnt_tot
    c_ref[...] = jnp.maximum(c_ref[...], c_tot)

    @pl.when(i == nblk - 1)
    def _epilogue():
        ng_f = ng_ref[...]
        scale_col = _row_to_col(jnp.exp(-c_ref[...]), _M)
        cnt_col = _row_to_col(cnt_ref[...], _M)
        mean_new = acc_ref[...] * scale_col
        cand = _MOM * ng_f + (1.0 - _MOM) * mean_new
        upd = jnp.where(cnt_col > 0, cand, ng_f)
        un = jnp.sqrt(jnp.sum(upd * upd, axis=1, keepdims=True))
        upd = upd / jnp.maximum(un, 1e-12)
        xx = upd + glove_ref[...]
        mu = jnp.mean(xx, axis=1, keepdims=True)
        var = jnp.mean((xx - mu) ** 2, axis=1, keepdims=True)
        xn = (xx - mu) / jnp.sqrt(var + 1e-5) * lnw_ref[...] + lnb_ref[...]
        h = jnp.dot(xn, w1_ref[...],
                    preferred_element_type=jnp.float32) + b1_ref[...]
        h = 0.5 * h * (1.0 + jax.lax.erf(h * (2.0 ** -0.5)))
        out_ref[...] = jnp.dot(h, w2_ref[...],
                               preferred_element_type=jnp.float32) + b2_ref[...]


def kernel(local_tokens, glove, ln_w, ln_b, W1, b1, W2, b2):
    n = local_tokens.shape[0] * local_tokens.shape[1]
    lf = local_tokens.reshape(n, _D)
    nblk = n // _TBLK
    rep = lambda i: (0, 0)
    return pl.pallas_call(
        _fused_kernel,
        grid=(nblk,),
        in_specs=[
            pl.BlockSpec((_TBLK, _D), lambda i: (i, 0)),
            pl.BlockSpec((_M, _D), rep),
            pl.BlockSpec((1, _D), rep),
            pl.BlockSpec((1, _D), rep),
            pl.BlockSpec((_D, _H), rep),
            pl.BlockSpec((1, _H), rep),
            pl.BlockSpec((_H, _D), rep),
            pl.BlockSpec((1, _D), rep),
        ],
        out_specs=pl.BlockSpec((_M, _D), rep),
        out_shape=jax.ShapeDtypeStruct((_M, _D), jnp.float32),
        scratch_shapes=[
            pltpu.VMEM((1, _M), jnp.float32),
            pltpu.VMEM((1, _M), jnp.float32),
            pltpu.VMEM((_M, _D), jnp.float32),
            pltpu.VMEM((_M, _D), jnp.float32),
        ],
    )(lf, glove, ln_w.reshape(1, _D), ln_b.reshape(1, _D),
      W1, b1.reshape(1, _H), W2, b2.reshape(1, _D))


# speedup vs baseline: 1.0695x; 1.0695x over previous
import jax
import jax.numpy as jnp
from jax.experimental import pallas as pl
from jax.experimental.pallas import tpu as pltpu

_M = 35
_D = 768
_H = _D // 2
_MOM = 0.8
_TBLK = 4096
_NCHUNK = 2


def _row_to_col(v, m):
    i0 = jax.lax.broadcasted_iota(jnp.int32, (m, m), 0)
    i1 = jax.lax.broadcasted_iota(jnp.int32, (m, m), 1)
    sel = jnp.where(i0 == i1, jnp.broadcast_to(v, (m, m)), 0.0)
    return jnp.sum(sel, axis=1, keepdims=True)


def _fused_kernel(x_ref, glove_ref, lnw_ref, lnb_ref, w1_ref, b1_ref,
                  w2_ref, b2_ref, out_ref, c_ref, cnt_ref, acc_ref, ng_ref):
    i = pl.program_id(0)
    nblk = pl.num_programs(0)

    @pl.when(i == 0)
    def _init():
        g = glove_ref[...]
        gn = jnp.sqrt(jnp.sum(g * g, axis=1, keepdims=True))
        ng_ref[...] = g / jnp.maximum(gn, 1e-12)
        c_ref[...] = jnp.full_like(c_ref, -2.0)
        cnt_ref[...] = jnp.zeros_like(cnt_ref)
        acc_ref[...] = jnp.zeros_like(acc_ref)

    ng = ng_ref[...]
    half = _TBLK // _NCHUNK
    accs, cnts, cs = [], [], []
    for h in range(_NCHUNK):
        x = x_ref[h * half:(h + 1) * half, :]
        ss = x[:, 0:128] * x[:, 0:128]
        for k in range(1, _D // 128):
            xs = x[:, k * 128:(k + 1) * 128]
            ss = ss + xs * xs
        sumsq = jnp.sum(ss, axis=1, keepdims=True)
        inv = jax.lax.rsqrt(jnp.maximum(sumsq, 1e-24))
        u = jax.lax.dot_general(x, ng, (((1,), (1,)), ((), ())),
                                preferred_element_type=jnp.float32)
        s = u * inv
        rowmax = jnp.max(s, axis=1, keepdims=True)
        onehot = jnp.where(s == rowmax, 1.0, 0.0)
        w = onehot * jnp.exp(rowmax)
        accs.append(jax.lax.dot_general(w, x, (((0,), (0,)), ((), ())),
                                        preferred_element_type=jnp.float32))
        cnts.append(jnp.sum(onehot, axis=0, keepdims=True))
        cs.append(jnp.max(s, axis=0, keepdims=True))
    acc_tot, cnt_tot, c_tot = accs[0], cnts[0], cs[0]
    for h in range(1, _NCHUNK):
        acc_tot = acc_tot + accs[h]
        cnt_tot = cnt_tot + cnts[h]
        c_tot = jnp.maximum(c_tot, cs[h])
    acc_ref[...] += acc_tot
    cnt_ref[...] += cnt_tot
    c_ref[...] = jnp.maximum(c_ref[...], c_tot)

    @pl.when(i == nblk - 1)
    def _epilogue():
        ng_f = ng_ref[...]
        scale_col = _row_to_col(jnp.exp(-c_ref[...]), _M)
        cnt_col = _row_to_col(cnt_ref[...], _M)
        mean_new = acc_ref[...] * scale_col
        cand = _MOM * ng_f + (1.0 - _MOM) * mean_new
        upd = jnp.where(cnt_col > 0, cand, ng_f)
        un = jnp.sqrt(jnp.sum(upd * upd, axis=1, keepdims=True))
        upd = upd / jnp.maximum(un, 1e-12)
        xx = upd + glove_ref[...]
        mu = jnp.mean(xx, axis=1, keepdims=True)
        var = jnp.mean((xx - mu) ** 2, axis=1, keepdims=True)
        xn = (xx - mu) / jnp.sqrt(var + 1e-5) * lnw_ref[...] + lnb_ref[...]
        h = jnp.dot(xn, w1_ref[...],
                    preferred_element_type=jnp.float32) + b1_ref[...]
        h = 0.5 * h * (1.0 + jax.lax.erf(h * (2.0 ** -0.5)))
        out_ref[...] = jnp.dot(h, w2_ref[...],
                               preferred_element_type=jnp.float32) + b2_ref[...]


def kernel(local_tokens, glove, ln_w, ln_b, W1, b1, W2, b2):
    n = local_tokens.shape[0] * local_tokens.shape[1]
    lf = local_tokens.reshape(n, _D)
    nblk = n // _TBLK
    rep = lambda i: (0, 0)
    return pl.pallas_call(
        _fused_kernel,
        grid=(nblk,),
        in_specs=[
            pl.BlockSpec((_TBLK, _D), lambda i: (i, 0)),
            pl.BlockSpec((_M, _D), rep),
            pl.BlockSpec((1, _D), rep),
            pl.BlockSpec((1, _D), rep),
            pl.BlockSpec((_D, _H), rep),
            pl.BlockSpec((1, _H), rep),
            pl.BlockSpec((_H, _D), rep),
            pl.BlockSpec((1, _D), rep),
        ],
        out_specs=pl.BlockSpec((_M, _D), rep),
        out_shape=jax.ShapeDtypeStruct((_M, _D), jnp.float32),
        scratch_shapes=[
            pltpu.VMEM((1, _M), jnp.float32),
            pltpu.VMEM((1, _M), jnp.float32),
            pltpu.VMEM((_M, _D), jnp.float32),
            pltpu.VMEM((_M, _D), jnp.float32),
        ],
    )(lf, glove, ln_w.reshape(1, _D), ln_b.reshape(1, _D),
      W1, b1.reshape(1, _H), W2, b2.reshape(1, _D))


# speedup vs baseline: 1.0899x; 1.0191x over previous
import jax
import jax.numpy as jnp
from jax.experimental import pallas as pl
from jax.experimental.pallas import tpu as pltpu

_M = 35
_D = 768
_H = _D // 2
_MOM = 0.8
_TBLK = 4096
_NCHUNK = 2


def _row_to_col(v, m):
    i0 = jax.lax.broadcasted_iota(jnp.int32, (m, m), 0)
    i1 = jax.lax.broadcasted_iota(jnp.int32, (m, m), 1)
    sel = jnp.where(i0 == i1, jnp.broadcast_to(v, (m, m)), 0.0)
    return jnp.sum(sel, axis=1, keepdims=True)


def _fused_kernel(x_ref, glove_ref, lnw_ref, lnb_ref, w1_ref, b1_ref,
                  w2_ref, b2_ref, out_ref, c_ref, acc_ref, ng_ref):
    i = pl.program_id(0)
    nblk = pl.num_programs(0)

    @pl.when(i == 0)
    def _init():
        g = glove_ref[...]
        gn = jnp.sqrt(jnp.sum(g * g, axis=1, keepdims=True))
        ng_ref[...] = g / jnp.maximum(gn, 1e-12)
        c_ref[...] = jnp.zeros_like(c_ref)
        acc_ref[...] = jnp.zeros_like(acc_ref)

    ng = ng_ref[...]
    half = _TBLK // _NCHUNK
    accs, cnts, cs = [], [], []
    for h in range(_NCHUNK):
        x = x_ref[h * half:(h + 1) * half, :]
        ss = x[:, 0:128] * x[:, 0:128]
        for k in range(1, _D // 128):
            xs = x[:, k * 128:(k + 1) * 128]
            ss = ss + xs * xs
        sumsq = jnp.sum(ss, axis=1, keepdims=True)
        inv = jax.lax.rsqrt(jnp.maximum(sumsq, 1e-24))
        u = jax.lax.dot_general(x, ng, (((1,), (1,)), ((), ())),
                                preferred_element_type=jnp.float32)
        rowmax = jnp.max(u, axis=1, keepdims=True)
        wtok = jnp.exp(rowmax * inv)
        w = jnp.where(u == rowmax, wtok, 0.0)
        accs.append(jax.lax.dot_general(w, x, (((0,), (0,)), ((), ())),
                                        preferred_element_type=jnp.float32))
        cs.append(jnp.max(w, axis=0, keepdims=True))
    acc_tot, c_tot = accs[0], cs[0]
    for h in range(1, _NCHUNK):
        acc_tot = acc_tot + accs[h]
        c_tot = jnp.maximum(c_tot, cs[h])
    acc_ref[...] += acc_tot
    c_ref[...] = jnp.maximum(c_ref[...], c_tot)

    @pl.when(i == nblk - 1)
    def _epilogue():
        ng_f = ng_ref[...]
        wmax_col = _row_to_col(c_ref[...], _M)
        scale_col = jnp.where(wmax_col > 0, 1.0 / wmax_col, 0.0)
        mean_new = acc_ref[...] * scale_col
        cand = _MOM * ng_f + (1.0 - _MOM) * mean_new
        upd = jnp.where(wmax_col > 0, cand, ng_f)
        un = jnp.sqrt(jnp.sum(upd * upd, axis=1, keepdims=True))
        upd = upd / jnp.maximum(un, 1e-12)
        xx = upd + glove_ref[...]
        mu = jnp.mean(xx, axis=1, keepdims=True)
        var = jnp.mean((xx - mu) ** 2, axis=1, keepdims=True)
        xn = (xx - mu) / jnp.sqrt(var + 1e-5) * lnw_ref[...] + lnb_ref[...]
        h = jnp.dot(xn, w1_ref[...],
                    preferred_element_type=jnp.float32) + b1_ref[...]
        h = 0.5 * h * (1.0 + jax.lax.erf(h * (2.0 ** -0.5)))
        out_ref[...] = jnp.dot(h, w2_ref[...],
                               preferred_element_type=jnp.float32) + b2_ref[...]


def kernel(local_tokens, glove, ln_w, ln_b, W1, b1, W2, b2):
    n = local_tokens.shape[0] * local_tokens.shape[1]
    lf = local_tokens.reshape(n, _D)
    nblk = n // _TBLK
    rep = lambda i: (0, 0)
    return pl.pallas_call(
        _fused_kernel,
        grid=(nblk,),
        in_specs=[
            pl.BlockSpec((_TBLK, _D), lambda i: (i, 0)),
            pl.BlockSpec((_M, _D), rep),
            pl.BlockSpec((1, _D), rep),
            pl.BlockSpec((1, _D), rep),
            pl.BlockSpec((_D, _H), rep),
            pl.BlockSpec((1, _H), rep),
            pl.BlockSpec((_H, _D), rep),
            pl.BlockSpec((1, _D), rep),
        ],
        out_specs=pl.BlockSpec((_M, _D), rep),
        out_shape=jax.ShapeDtypeStruct((_M, _D), jnp.float32),
        scratch_shapes=[
            pltpu.VMEM((1, _M), jnp.float32),
            pltpu.VMEM((_M, _D), jnp.float32),
            pltpu.VMEM((_M, _D), jnp.float32),
        ],
    )(lf, glove, ln_w.reshape(1, _D), ln_b.reshape(1, _D),
      W1, b1.reshape(1, _H), W2, b2.reshape(1, _D))


# speedup vs baseline: 1.0944x; 1.0041x over previous
import jax
import jax.numpy as jnp
from jax.experimental import pallas as pl
from jax.experimental.pallas import tpu as pltpu

_M = 35
_D = 768
_H = _D // 2
_MOM = 0.8
_TBLK = 4096
_NCHUNK = 2


def _row_to_col(v, m):
    i0 = jax.lax.broadcasted_iota(jnp.int32, (m, m), 0)
    i1 = jax.lax.broadcasted_iota(jnp.int32, (m, m), 1)
    sel = jnp.where(i0 == i1, jnp.broadcast_to(v, (m, m)), 0.0)
    return jnp.sum(sel, axis=1, keepdims=True)


def _fused_kernel(x_ref, glove_ref, lnw_ref, lnb_ref, w1_ref, b1_ref,
                  w2_ref, b2_ref, out_ref, c_ref, acc_ref, ng_ref):
    i = pl.program_id(0)
    nblk = pl.num_programs(0)

    @pl.when(i == 0)
    def _init():
        g = glove_ref[...]
        gn = jnp.sqrt(jnp.sum(g * g, axis=1, keepdims=True))
        ng_ref[...] = g / jnp.maximum(gn, 1e-12)
        c_ref[...] = jnp.zeros_like(c_ref)
        acc_ref[...] = jnp.zeros_like(acc_ref)

    ng = ng_ref[...].astype(jnp.bfloat16)
    half = _TBLK // _NCHUNK
    accs, cnts, cs = [], [], []
    for h in range(_NCHUNK):
        x = x_ref[h * half:(h + 1) * half, :]
        xb = x.astype(jnp.bfloat16)
        ss = xb[:, 0:128] * xb[:, 0:128]
        for k in range(1, _D // 128):
            xs = xb[:, k * 128:(k + 1) * 128]
            ss = ss + xs * xs
        sumsq = jnp.sum(ss.astype(jnp.float32), axis=1,
                        keepdims=True)
        inv = jax.lax.rsqrt(jnp.maximum(sumsq, 1e-24))
        u = jax.lax.dot_general(xb, ng, (((1,), (1,)), ((), ())),
                                preferred_element_type=jnp.float32)
        rowmax = jnp.max(u, axis=1, keepdims=True)
        wtok = jnp.exp(rowmax * inv)
        w = jnp.where(u == rowmax, wtok, 0.0)
        accs.append(jax.lax.dot_general(w, xb, (((0,), (0,)), ((), ())),
                                        preferred_element_type=jnp.float32))
        cs.append(jnp.max(w, axis=0, keepdims=True))
    acc_tot, c_tot = accs[0], cs[0]
    for h in range(1, _NCHUNK):
        acc_tot = acc_tot + accs[h]
        c_tot = jnp.maximum(c_tot, cs[h])
    acc_ref[...] += acc_tot
    c_ref[...] = jnp.maximum(c_ref[...], c_tot)

    @pl.when(i == nblk - 1)
    def _epilogue():
        ng_f = ng_ref[...]
        wmax_col = _row_to_col(c_ref[...], _M)
        scale_col = jnp.where(wmax_col > 0, 1.0 / wmax_col, 0.0)
        mean_new = acc_ref[...] * scale_col
        cand = _MOM * ng_f + (1.0 - _MOM) * mean_new
        upd = jnp.where(wmax_col > 0, cand, ng_f)
        un = jnp.sqrt(jnp.sum(upd * upd, axis=1, keepdims=True))
        upd = upd / jnp.maximum(un, 1e-12)
        xx = upd + glove_ref[...]
        mu = jnp.mean(xx, axis=1, keepdims=True)
        var = jnp.mean((xx - mu) ** 2, axis=1, keepdims=True)
        xn = (xx - mu) / jnp.sqrt(var + 1e-5) * lnw_ref[...] + lnb_ref[...]
        h = jnp.dot(xn, w1_ref[...],
                    preferred_element_type=jnp.float32) + b1_ref[...]
        h = 0.5 * h * (1.0 + jax.lax.erf(h * (2.0 ** -0.5)))
        out_ref[...] = jnp.dot(h, w2_ref[...],
                               preferred_element_type=jnp.float32) + b2_ref[...]


def kernel(local_tokens, glove, ln_w, ln_b, W1, b1, W2, b2):
    n = local_tokens.shape[0] * local_tokens.shape[1]
    lf = local_tokens.reshape(n, _D)
    nblk = n // _TBLK
    rep = lambda i: (0, 0)
    return pl.pallas_call(
        _fused_kernel,
        grid=(nblk,),
        in_specs=[
            pl.BlockSpec((_TBLK, _D), lambda i: (i, 0)),
            pl.BlockSpec((_M, _D), rep),
            pl.BlockSpec((1, _D), rep),
            pl.BlockSpec((1, _D), rep),
            pl.BlockSpec((_D, _H), rep),
            pl.BlockSpec((1, _H), rep),
            pl.BlockSpec((_H, _D), rep),
            pl.BlockSpec((1, _D), rep),
        ],
        out_specs=pl.BlockSpec((_M, _D), rep),
        out_shape=jax.ShapeDtypeStruct((_M, _D), jnp.float32),
        scratch_shapes=[
            pltpu.VMEM((1, _M), jnp.float32),
            pltpu.VMEM((_M, _D), jnp.float32),
            pltpu.VMEM((_M, _D), jnp.float32),
        ],
    )(lf, glove, ln_w.reshape(1, _D), ln_b.reshape(1, _D),
      W1, b1.reshape(1, _H), W2, b2.reshape(1, _D))
